# baseline (speedup 1.0000x reference)
import math

import jax
import jax.numpy as jnp
import numpy as np
from jax.experimental import pallas as pl
from jax.experimental.pallas import tpu as pltpu

_E = 128
_H = 4
_DH = 32
_OUT = 768
_CB = 1024
_NCHUNK = 2
_B = _CB * _NCHUNK
_SCALE = 1.0 / math.sqrt(_DH)

_OFF_RV = 0
_OFF_B = 128
_OFF_SELF = 136
_OFF_TV1 = 144
_OFF_TV0 = 152
_OFF_VT1 = 160
_OFF_VT0 = 168
_OFF_PAIR = 176

_PAIRS = (
    ('p', 'd', 176, 216),
    ('a', 'd', 176, 216),
    ('a', 'p', 256, 296),
    ('a', 'u', 336, 416),
    ('u', 'd', 176, 216),
    ('u', 'p', 256, 296),
)
_TOKS = ('d', 'p', 'a', 'v', 'u')
_VOCAB = {'d': 10, 'p': 10, 'a': 100, 'u': 20}
_PERM = np.arange(128)
_PERM = (_PERM % 4) * 32 + _PERM // 4


def kernel(device_ids, pseudo_ids, attr_ids, unit_ids, values, mask,
           dev_table, pseudo_table, attr_table, unit_table,
           val_w, val_b, in_proj_w, in_proj_b, out_proj_w, out_proj_b,
           out_w, out_b):
    n = device_ids.shape[0]
    B = _B
    G = n // B
    f32 = jnp.float32

    ids_in = {
        'd': device_ids.astype(jnp.int32).reshape(G, 1, B),
        'p': pseudo_ids.astype(jnp.int32).reshape(G, 1, B),
        'a': attr_ids.astype(jnp.int32).reshape(G, 1, B),
        'u': unit_ids.astype(jnp.int32).reshape(G, 1, B),
    }
    vals3 = values.astype(f32).reshape(G, 1, B)
    mask3 = mask.astype(jnp.int32).T.reshape(5, G, B).transpose(1, 0, 2)

    W = in_proj_w.astype(f32)
    bias3 = in_proj_b.astype(f32)
    bv = bias3[2 * _E:]
    t_cat = jnp.concatenate(
        [dev_table, pseudo_table, attr_table, unit_table], axis=0
    ).astype(f32)
    extra = jnp.stack([val_w[:, 0].astype(f32), val_b.astype(f32)])
    rqkv = jnp.concatenate([t_cat, extra], axis=0) @ W.T
    q_all = jnp.concatenate([rqkv[:, :_E], bias3[None, :_E]], axis=0)
    k_all = jnp.concatenate([rqkv[:, _E:2 * _E], bias3[None, _E:2 * _E]],
                            axis=0)
    s_all = jnp.einsum('ihd,jhd->hij',
                       q_all.reshape(143, _H, _DH),
                       k_all.reshape(143, _H, _DH)) * _SCALE
    v_all = rqkv[:, 2 * _E:]
    v_perm = v_all.T[_PERM]

    _QOFF = {'d': 0, 'p': 10, 'a': 20, 'u': 120}
    _IVW, _IVB, _IBIAS = 140, 141, 142

    def _idx(t):
        R = _VOCAB[t]
        o = _QOFF[t]
        c = np.arange(R)[None, :]
        rows = []

        def blk8(i_of_c, j_of_c):
            s = np.arange(8)[:, None]
            h = s % 4
            i = np.broadcast_to(i_of_c, (8, R))
            j = np.broadcast_to(j_of_c, (8, R))
            rows.append((h * np.ones((8, R), np.int64), i, j))

        blk8(_IBIAS, o + c)
        blk8(o + c, o + c)
        blk8(o + c, _IVW)
        blk8(o + c, _IVB)
        blk8(_IVW, o + c)
        blk8(_IVB, o + c)
        for gt, st, _, _ in _PAIRS:
            if gt != t:
                continue
            Rs = _VOCAB[st]
            p = np.arange(4 * Rs)[:, None]
            h = np.broadcast_to(p % 4, (4 * Rs, R))
            si = np.broadcast_to(_QOFF[st] + p // 4, (4 * Rs, R))
            gi = np.broadcast_to(c, (4 * Rs, R))
            rows.append((h, si, o + gi))
            rows.append((h, o + gi, si))
        hh = np.concatenate([r[0] for r in rows], axis=0)
        ii = np.concatenate([r[1] for r in rows], axis=0)
        jj = np.concatenate([r[2] for r in rows], axis=0)
        return hh, ii, jj

    tab = {}
    for t in _VOCAB:
        hh, ii, jj = _idx(t)
        R = _VOCAB[t]
        o = _QOFF[t]
        tab[t] = jnp.concatenate(
            [v_perm[:, o:o + R], s_all[hh, ii, jj]], axis=0)

    h8 = np.arange(8) % 4
    vc_i = np.array([_IVW] * 8 + [_IVW] * 8 + [_IVB] * 8
                    + [_IBIAS] * 8 + [_IBIAS] * 8)
    vc_j = np.array([_IVW] * 8 + [_IVB] * 8 + [_IVB] * 8
                    + [_IVW] * 8 + [_IVB] * 8)
    vc_col = s_all[np.tile(h8, 5), vc_i, vc_j]
    vc_col = vc_col.at[8:16].add(s_all[h8, _IVB, _IVW])
    vc = jnp.broadcast_to(vc_col[:, None], (40, _CB))
    vwv_b = jnp.broadcast_to(v_perm[:, _IVW][:, None], (_E, _CB))
    vbv_b = jnp.broadcast_to(v_perm[:, _IVB][:, None], (_E, _CB))

    w2 = out_proj_w.T.astype(f32) @ out_w.T.astype(f32)
    b2 = out_proj_b @ out_w.T + out_b + bv @ w2
    w2ext = jnp.concatenate(
        [w2[_PERM], b2[None, :], jnp.zeros((7, _OUT), f32)], axis=0)

    def _body(idd_ref, idp_ref, ida_ref, idu_ref, vals_ref, mask_ref,
              td_ref, tp_ref, ta_ref, tu_ref,
              vc_ref, vwv_ref, vbv_ref, w2_ref, out_ref):
        idrefs = {'d': idd_ref, 'p': idp_ref, 'a': ida_ref, 'u': idu_ref}
        trefs = {'d': td_ref, 'p': tp_ref, 'a': ta_ref, 'u': tu_ref}
        vcb = vc_ref[...]

        def do_chunk(c):
            sl = slice(c * _CB, (c + 1) * _CB)
            ids = {t: idrefs[t][0][:, sl] for t in idrefs}
            vals = vals_ref[0][:, sl]
            mk = mask_ref[0][:, sl].astype(f32)

            g = {}
            for t in ('d', 'p', 'a', 'u'):
                R = _VOCAB[t]
                io = jax.lax.broadcasted_iota(jnp.int32, (R, _CB), 0)
                oh = jnp.where(io == ids[t], 1.0, 0.0)
                g[t] = jnp.dot(trefs[t][...], oh, preferred_element_type=f32)

            ohe = {}
            for t in ('d', 'p', 'u'):
                R = _VOCAB[t]
                io4 = jax.lax.broadcasted_iota(
                    jnp.int32, (_H * R, _CB), 0) // _H
                ohe[t] = jnp.where(io4 == ids[t], 1.0, 0.0)

            mk8 = {t: jnp.broadcast_to(mk[i:i + 1], (8, _CB))
                   for i, t in enumerate(_TOKS)}
            val8 = jnp.broadcast_to(vals, (8, _CB))

            def seg_reduce(prod):
                s = prod[0:8]
                for k in range(1, prod.shape[0] // 8):
                    s = s + prod[8 * k:8 * (k + 1)]
                return s + jnp.concatenate([s[4:8], s[0:4]], axis=0)

            P = {}
            B8 = {}
            for t in ('d', 'p', 'a', 'u'):
                B8[t] = g[t][_OFF_B:_OFF_B + 8]
                P[(t, t)] = g[t][_OFF_SELF:_OFF_SELF + 8]
                P[(t, 'v')] = g[t][_OFF_TV1:_OFF_TV1 + 8] * val8 \
                    + g[t][_OFF_TV0:_OFF_TV0 + 8]
                P[('v', t)] = g[t][_OFF_VT1:_OFF_VT1 + 8] * val8 \
                    + g[t][_OFF_VT0:_OFF_VT0 + 8]
            B8['v'] = vcb[24:32] * val8 + vcb[32:40]
            P[('v', 'v')] = (vcb[0:8] * val8 + vcb[8:16]) * val8 + vcb[16:24]
            for gt, st, off1, off2 in _PAIRS:
                w1 = _H * _VOCAB[st]
                P[(st, gt)] = seg_reduce(g[gt][off1:off1 + w1] * ohe[st])
                P[(gt, st)] = seg_reduce(g[gt][off2:off2 + w1] * ohe[st])

            wsum = {t: None for t in _TOKS}
            for tq in _TOKS:
                ls = [mk8[tk] * (mk8[tq] * P[(tq, tk)] + B8[tk])
                      for tk in _TOKS]
                m = jnp.maximum(jnp.maximum(jnp.maximum(ls[0], ls[1]),
                                            jnp.maximum(ls[2], ls[3])), ls[4])
                e = [jnp.exp(x - m) for x in ls]
                r = 1.0 / (e[0] + e[1] + e[2] + e[3] + e[4])
                for i, tk in enumerate(_TOKS):
                    w = e[i] * r
                    wsum[tk] = w if wsum[tk] is None else wsum[tk] + w

            o = None
            for tk in _TOKS:
                wm = wsum[tk] * mk8[tk] * (1.0 / len(_TOKS))
                wf = jnp.concatenate([wm] * (_E // 8), axis=0)
                if tk == 'v':
                    rv_tok = vwv_ref[...] * jnp.concatenate(
                        [val8] * (_E // 8), axis=0) + vbv_ref[...]
                else:
                    rv_tok = g[tk][_OFF_RV:_OFF_RV + _E]
                ctr = wf * rv_tok
                o = ctr if o is None else o + ctr

            acc = jnp.concatenate([o, jnp.ones((8, _CB), f32)],
                                  axis=0)
            out_ref[sl, :] = jax.lax.dot_general(
                acc, w2_ref[...],
                dimension_numbers=(((0,), (0,)), ((), ())),
                preferred_element_type=f32)

        for c in range(_NCHUNK):
            do_chunk(c)

    return pl.pallas_call(
        _body,
        grid=(G,),
        in_specs=[
            pl.BlockSpec((1, 1, B), lambda i: (i, 0, 0)),
            pl.BlockSpec((1, 1, B), lambda i: (i, 0, 0)),
            pl.BlockSpec((1, 1, B), lambda i: (i, 0, 0)),
            pl.BlockSpec((1, 1, B), lambda i: (i, 0, 0)),
            pl.BlockSpec((1, 1, B), lambda i: (i, 0, 0)),
            pl.BlockSpec((1, 5, B), lambda i: (i, 0, 0)),
            pl.BlockSpec(tab['d'].shape, lambda i: (0, 0)),
            pl.BlockSpec(tab['p'].shape, lambda i: (0, 0)),
            pl.BlockSpec(tab['a'].shape, lambda i: (0, 0)),
            pl.BlockSpec(tab['u'].shape, lambda i: (0, 0)),
            pl.BlockSpec((40, _CB), lambda i: (0, 0)),
            pl.BlockSpec((_E, _CB), lambda i: (0, 0)),
            pl.BlockSpec((_E, _CB), lambda i: (0, 0)),
            pl.BlockSpec((136, _OUT), lambda i: (0, 0)),
        ],
        out_specs=pl.BlockSpec((B, _OUT), lambda i: (i, 0)),
        out_shape=jax.ShapeDtypeStruct((n, _OUT), f32),
        compiler_params=pltpu.CompilerParams(
            dimension_semantics=("parallel",),
            vmem_limit_bytes=56 * 1024 * 1024,
        ),
    )(ids_in['d'], ids_in['p'], ids_in['a'], ids_in['u'], vals3, mask3,
      tab['d'], tab['p'], tab['a'], tab['u'], vc, vwv_b, vbv_b, w2ext)


# speedup vs baseline: 1.0366x; 1.0366x over previous
import math

import jax
import jax.numpy as jnp
import numpy as np
from jax.experimental import pallas as pl
from jax.experimental.pallas import tpu as pltpu

_E = 128
_H = 4
_DH = 32
_OUT = 768
_CB = 1024
_NCHUNK = 2
_B = _CB * _NCHUNK
_SCALE = 1.0 / math.sqrt(_DH)

_OFF_RV = 0
_OFF_B = 128
_OFF_SELF = 136
_OFF_TV1 = 144
_OFF_TV0 = 152
_OFF_VT1 = 160
_OFF_VT0 = 168
_OFF_PAIR = 176

_PAIRS = (
    ('p', 'd', 176, 216),
    ('a', 'd', 176, 216),
    ('a', 'p', 256, 296),
    ('a', 'u', 336, 416),
    ('u', 'd', 176, 216),
    ('u', 'p', 256, 296),
)
_TOKS = ('d', 'p', 'a', 'v', 'u')
_VOCAB = {'d': 10, 'p': 10, 'a': 100, 'u': 20}
_PERM = np.arange(128)
_PERM = (_PERM % 4) * 32 + _PERM // 4


def kernel(device_ids, pseudo_ids, attr_ids, unit_ids, values, mask,
           dev_table, pseudo_table, attr_table, unit_table,
           val_w, val_b, in_proj_w, in_proj_b, out_proj_w, out_proj_b,
           out_w, out_b):
    n = device_ids.shape[0]
    B = _B
    G = n // B
    f32 = jnp.float32

    ids_in = {
        'd': device_ids.astype(jnp.int32).reshape(G, 1, B),
        'p': pseudo_ids.astype(jnp.int32).reshape(G, 1, B),
        'a': attr_ids.astype(jnp.int32).reshape(G, 1, B),
        'u': unit_ids.astype(jnp.int32).reshape(G, 1, B),
    }
    vals3 = values.astype(f32).reshape(G, 1, B)
    mask3 = mask.astype(jnp.int32).T.reshape(5, G, B).transpose(1, 0, 2)

    W = in_proj_w.astype(f32)
    Wq, Wk, Wv = W[:_E], W[_E:2 * _E], W[2 * _E:]
    bq, bk, bv = (in_proj_b[:_E].astype(f32), in_proj_b[_E:2 * _E].astype(f32),
                  in_proj_b[2 * _E:].astype(f32))
    tabs = {'d': dev_table, 'p': pseudo_table, 'a': attr_table, 'u': unit_table}
    rq = {t: tabs[t].astype(f32) @ Wq.T for t in tabs}
    rk = {t: tabs[t].astype(f32) @ Wk.T for t in tabs}
    rv = {t: tabs[t].astype(f32) @ Wv.T for t in tabs}
    vvec = val_w[:, 0].astype(f32)
    vb0 = val_b.astype(f32)
    vwq, vwk, vwv = Wq @ vvec, Wk @ vvec, Wv @ vvec
    vbq, vbk, vbv = Wq @ vb0, Wk @ vb0, Wv @ vb0

    def _hd(a, b):
        p = a * b
        return p.reshape(*p.shape[:-1], _H, _DH).sum(-1) * _SCALE

    def _il8(x):
        xt = jnp.moveaxis(x, -1, 0)
        return jnp.concatenate([xt, xt], axis=0)

    def _pair_block(qa, kb):
        ps = _hd(qa[:, None, :], kb[None, :, :])
        return ps.transpose(0, 2, 1).reshape(qa.shape[0] * _H, kb.shape[0])

    blocks_for = {t: [] for t in tabs}
    for g, s, _, _ in _PAIRS:
        blocks_for[g].append(_pair_block(rq[s], rk[g]))
        blocks_for[g].append(_pair_block(rk[s], rq[g]))

    def build_table(t):
        R = _VOCAB[t]
        rows = [rv[t].T[_PERM],
                _il8(_hd(rk[t], bq)).reshape(8, R),
                _il8(_hd(rq[t], rk[t])).reshape(8, R),
                _il8(_hd(rq[t], vwk)).reshape(8, R),
                _il8(_hd(rq[t], vbk)).reshape(8, R),
                _il8(_hd(rk[t], vwq)).reshape(8, R),
                _il8(_hd(rk[t], vbq)).reshape(8, R)]
        rows += blocks_for[t]
        return jnp.concatenate(rows, axis=0)

    tab = {t: build_table(t) for t in tabs}

    vc_col = jnp.concatenate([
        _il8(_hd(vwq, vwk)),
        _il8(_hd(vwq, vbk) + _hd(vbq, vwk)),
        _il8(_hd(vbq, vbk)),
        _il8(_hd(bq, vwk)),
        _il8(_hd(bq, vbk)),
    ])
    vc = jnp.broadcast_to(vc_col[:, None], (40, _CB))
    vwv_b = jnp.broadcast_to(vwv[_PERM][:, None], (_E, _CB))
    vbv_b = jnp.broadcast_to(vbv[_PERM][:, None], (_E, _CB))

    w2 = out_proj_w.T.astype(f32) @ out_w.T.astype(f32)
    b2 = out_proj_b @ out_w.T + out_b + bv @ w2
    w2ext = jnp.concatenate(
        [w2[_PERM], b2[None, :], jnp.zeros((7, _OUT), f32)], axis=0)

    def _body(idd_ref, idp_ref, ida_ref, idu_ref, vals_ref, mask_ref,
              td_ref, tp_ref, ta_ref, tu_ref,
              vc_ref, vwv_ref, vbv_ref, w2_ref, out_ref):
        idrefs = {'d': idd_ref, 'p': idp_ref, 'a': ida_ref, 'u': idu_ref}
        trefs = {'d': td_ref, 'p': tp_ref, 'a': ta_ref, 'u': tu_ref}
        vcb = vc_ref[...]

        def do_chunk(c):
            sl = slice(c * _CB, (c + 1) * _CB)
            ids = {t: idrefs[t][0][:, sl] for t in idrefs}
            vals = vals_ref[0][:, sl]
            mk = mask_ref[0][:, sl].astype(f32)

            g = {}
            for t in ('d', 'p', 'a', 'u'):
                R = _VOCAB[t]
                io = jax.lax.broadcasted_iota(jnp.int32, (R, _CB), 0)
                oh = jnp.where(io == ids[t], 1.0, 0.0)
                g[t] = jnp.dot(trefs[t][...], oh, preferred_element_type=f32)

            ohe = {}
            for t in ('d', 'p', 'u'):
                R = _VOCAB[t]
                io4 = jax.lax.broadcasted_iota(
                    jnp.int32, (_H * R, _CB), 0) // _H
                ohe[t] = jnp.where(io4 == ids[t], 1.0, 0.0)

            mk8 = {t: jnp.broadcast_to(mk[i:i + 1], (8, _CB))
                   for i, t in enumerate(_TOKS)}
            val8 = jnp.broadcast_to(vals, (8, _CB))

            def seg_reduce(prod):
                s = prod[0:8]
                for k in range(1, prod.shape[0] // 8):
                    s = s + prod[8 * k:8 * (k + 1)]
                return s + jnp.concatenate([s[4:8], s[0:4]], axis=0)

            P = {}
            B8 = {}
            for t in ('d', 'p', 'a', 'u'):
                B8[t] = g[t][_OFF_B:_OFF_B + 8]
                P[(t, t)] = g[t][_OFF_SELF:_OFF_SELF + 8]
                P[(t, 'v')] = g[t][_OFF_TV1:_OFF_TV1 + 8] * val8 \
                    + g[t][_OFF_TV0:_OFF_TV0 + 8]
                P[('v', t)] = g[t][_OFF_VT1:_OFF_VT1 + 8] * val8 \
                    + g[t][_OFF_VT0:_OFF_VT0 + 8]
            B8['v'] = vcb[24:32] * val8 + vcb[32:40]
            P[('v', 'v')] = (vcb[0:8] * val8 + vcb[8:16]) * val8 + vcb[16:24]
            for gt, st, off1, off2 in _PAIRS:
                w1 = _H * _VOCAB[st]
                P[(st, gt)] = seg_reduce(g[gt][off1:off1 + w1] * ohe[st])
                P[(gt, st)] = seg_reduce(g[gt][off2:off2 + w1] * ohe[st])

            wsum = {t: None for t in _TOKS}
            for tq in _TOKS:
                ls = [mk8[tk] * (mk8[tq] * P[(tq, tk)] + B8[tk])
                      for tk in _TOKS]
                m = jnp.maximum(jnp.maximum(jnp.maximum(ls[0], ls[1]),
                                            jnp.maximum(ls[2], ls[3])), ls[4])
                e = [jnp.exp(x - m) for x in ls]
                r = 1.0 / (e[0] + e[1] + e[2] + e[3] + e[4])
                for i, tk in enumerate(_TOKS):
                    w = e[i] * r
                    wsum[tk] = w if wsum[tk] is None else wsum[tk] + w

            o = None
            for tk in _TOKS:
                wm = wsum[tk] * mk8[tk] * (1.0 / len(_TOKS))
                wf = jnp.concatenate([wm] * (_E // 8), axis=0)
                if tk == 'v':
                    rv_tok = vwv_ref[...] * jnp.concatenate(
                        [val8] * (_E // 8), axis=0) + vbv_ref[...]
                else:
                    rv_tok = g[tk][_OFF_RV:_OFF_RV + _E]
                ctr = wf * rv_tok
                o = ctr if o is None else o + ctr

            acc = jnp.concatenate([o, jnp.ones((8, _CB), f32)],
                                  axis=0)
            out_ref[sl, :] = jax.lax.dot_general(
                acc, w2_ref[...],
                dimension_numbers=(((0,), (0,)), ((), ())),
                preferred_element_type=f32)

        for c in range(_NCHUNK):
            do_chunk(c)

    return pl.pallas_call(
        _body,
        grid=(G,),
        in_specs=[
            pl.BlockSpec((1, 1, B), lambda i: (i, 0, 0)),
            pl.BlockSpec((1, 1, B), lambda i: (i, 0, 0)),
            pl.BlockSpec((1, 1, B), lambda i: (i, 0, 0)),
            pl.BlockSpec((1, 1, B), lambda i: (i, 0, 0)),
            pl.BlockSpec((1, 1, B), lambda i: (i, 0, 0)),
            pl.BlockSpec((1, 5, B), lambda i: (i, 0, 0)),
            pl.BlockSpec(tab['d'].shape, lambda i: (0, 0)),
            pl.BlockSpec(tab['p'].shape, lambda i: (0, 0)),
            pl.BlockSpec(tab['a'].shape, lambda i: (0, 0)),
            pl.BlockSpec(tab['u'].shape, lambda i: (0, 0)),
            pl.BlockSpec((40, _CB), lambda i: (0, 0)),
            pl.BlockSpec((_E, _CB), lambda i: (0, 0)),
            pl.BlockSpec((_E, _CB), lambda i: (0, 0)),
            pl.BlockSpec((136, _OUT), lambda i: (0, 0)),
        ],
        out_specs=pl.BlockSpec((B, _OUT), lambda i: (i, 0)),
        out_shape=jax.ShapeDtypeStruct((n, _OUT), f32),
        compiler_params=pltpu.CompilerParams(
            dimension_semantics=("parallel",),
            vmem_limit_bytes=56 * 1024 * 1024,
        ),
    )(ids_in['d'], ids_in['p'], ids_in['a'], ids_in['u'], vals3, mask3,
      tab['d'], tab['p'], tab['a'], tab['u'], vc, vwv_b, vbv_b, w2ext)
